# Initial kernel scaffold; baseline (speedup 1.0000x reference)
import jax
import jax.numpy as jnp
from jax.experimental import pallas as pl


def kernel(x, edge_index, W1_0, b1_0, W2_0, b2_0, W1_1, b1_1, W2_1, b2_1, W1_2, b1_2, W2_2, b2_2, W1_3, b1_3, W2_3, b2_3, Wout, bout):
    raise NotImplementedError("write your pallas kernel here")


# speedup vs baseline: 3.3397x; 3.3397x over previous
import functools

import jax
import jax.numpy as jnp
from jax import lax
from jax.experimental import pallas as pl
from jax.experimental.pallas import tpu as pltpu
from jax.experimental.pallas import tpu_sc as plsc

D = 128
NNODES = 10000
NP = 10240
NEDGES = 320000
C = 128
NW = 32
CPW = (NEDGES + NW * C - 1) // (NW * C)
EP = NW * C * CPW
RPT = NP // 16
BLK = 1024
DEGW = 16


@functools.cache
def _sc_kernels():
    mesh = plsc.VectorSubcoreMesh(core_axis_name="c", subcore_axis_name="s")

    @functools.partial(
        pl.kernel,
        mesh=mesh,
        out_type=jax.ShapeDtypeStruct((2, NP, D), jnp.float32),
        scratch_types=[
            pltpu.VMEM((C,), jnp.int32),
            pltpu.VMEM((C,), jnp.int32),
            pltpu.VMEM((C, D), jnp.float32),
            pltpu.VMEM((C, D), jnp.float32),
            pltpu.VMEM_SHARED((NP, D), jnp.float32),
            pltpu.SemaphoreType.DMA,
            pltpu.SemaphoreType.DMA,
        ],
    )
    def sc_edge_layer(src_hbm, dst_hbm, a_hbm, b_hbm, out_hbm,
                      dsti, srci, arows, brows, acc, sem_a, sem_b):
        cid = lax.axis_index("c")
        sid = lax.axis_index("s")
        wid = cid * 16 + sid

        def zrow(i, _):
            for j in range(D // 16):
                brows[i, pl.ds(j * 16, 16)] = jnp.zeros((16,), jnp.float32)
            return 0
        lax.fori_loop(0, C, zrow, 0)
        for k in range(RPT // C):
            pltpu.sync_copy(brows, acc.at[pl.ds(sid * RPT + k * C, C)])
        plsc.subcore_barrier()

        def chunk(k, _):
            base = pl.multiple_of((wid * CPW + k) * C, 8)
            pltpu.sync_copy(dst_hbm.at[pl.ds(base, C)], dsti)
            pltpu.sync_copy(src_hbm.at[pl.ds(base, C)], srci)
            cp_a = pltpu.async_copy(a_hbm.at[dsti], arows, sem_a)
            cp_b = pltpu.async_copy(b_hbm.at[srci], brows, sem_b)
            cp_a.wait()
            cp_b.wait()

            def row(e, _):
                for j in range(D // 16):
                    xv = (arows[e, pl.ds(j * 16, 16)]
                          + brows[e, pl.ds(j * 16, 16)])
                    tv = 1.0 - 2.0 / (1.0 + jnp.exp(xv + xv))
                    ti = lax.bitcast_convert_type(tv, jnp.int32)
                    ti = (ti + 32767 + ((ti >> 16) & 1)) & jnp.int32(-65536)
                    arows[e, pl.ds(j * 16, 16)] = lax.bitcast_convert_type(
                        ti, jnp.float32)
                return 0
            lax.fori_loop(0, C, row, 0)
            pltpu.sync_copy(arows, acc.at[dsti], add=True)
            return 0
        lax.fori_loop(0, CPW, chunk, 0)
        plsc.subcore_barrier()

        for k in range(RPT // C):
            r0 = sid * RPT + k * C
            pltpu.sync_copy(acc.at[pl.ds(r0, C)], arows)
            pltpu.sync_copy(arows, out_hbm.at[cid, pl.ds(r0, C)])

    @functools.partial(
        pl.kernel,
        mesh=mesh,
        out_type=jax.ShapeDtypeStruct((2, NP, DEGW), jnp.float32),
        scratch_types=[
            pltpu.VMEM((C,), jnp.int32),
            pltpu.VMEM((C, DEGW), jnp.float32),
            pltpu.VMEM((C, DEGW), jnp.float32),
            pltpu.VMEM_SHARED((NP, DEGW), jnp.float32),
        ],
    )
    def sc_degree(dst_hbm, out_hbm, dsti, ones, zbuf, acc):
        cid = lax.axis_index("c")
        sid = lax.axis_index("s")
        wid = cid * 16 + sid

        def fill(i, _):
            ones[i, pl.ds(0, 16)] = jnp.ones((16,), jnp.float32)
            zbuf[i, pl.ds(0, 16)] = jnp.zeros((16,), jnp.float32)
            return 0
        lax.fori_loop(0, C, fill, 0)
        for k in range(RPT // C):
            pltpu.sync_copy(zbuf, acc.at[pl.ds(sid * RPT + k * C, C)])
        plsc.subcore_barrier()

        def chunk(k, _):
            base = pl.multiple_of((wid * CPW + k) * C, 8)
            pltpu.sync_copy(dst_hbm.at[pl.ds(base, C)], dsti)
            pltpu.sync_copy(ones, acc.at[dsti], add=True)
            return 0
        lax.fori_loop(0, CPW, chunk, 0)
        plsc.subcore_barrier()

        for k in range(RPT // C):
            r0 = sid * RPT + k * C
            pltpu.sync_copy(acc.at[pl.ds(r0, C)], zbuf)
            pltpu.sync_copy(zbuf, out_hbm.at[cid, pl.ds(r0, C)])

    return sc_edge_layer, sc_degree


def _dot(a, b):
    return jnp.dot(a, b, preferred_element_type=jnp.float32,
                   precision=lax.Precision.HIGHEST)


def _bfr(v):
    return v.astype(jnp.bfloat16).astype(jnp.float32)


def _tc_pre_body(x_ref, w1_ref, b1_ref, a_ref, b_ref):
    w1 = _bfr(w1_ref[...])
    wb = w1[D:]
    wa = w1[:D] - wb
    xb = _bfr(x_ref[...])
    a_ref[...] = _dot(xb, wa) + b1_ref[...]
    b_ref[...] = _dot(xb, wb)


def _tc_mid_body(tp_ref, dp_ref, w2_ref, b2_ref, w1n_ref, b1n_ref, a_ref, b_ref):
    t = tp_ref[0] + tp_ref[1]
    d = (dp_ref[0] + dp_ref[1])[:, 0:1]
    h = _bfr(_dot(t, _bfr(w2_ref[...])) + d * b2_ref[...])
    w1n = _bfr(w1n_ref[...])
    wb = w1n[D:]
    wa = w1n[:D] - wb
    a_ref[...] = _dot(h, wa) + b1n_ref[...]
    b_ref[...] = _dot(h, wb)


def _tc_fin_body(tp_ref, dp_ref, w2_ref, b2_ref, wo_ref, bo_ref, o_ref):
    t = tp_ref[0] + tp_ref[1]
    d = (dp_ref[0] + dp_ref[1])[:, 0:1]
    h = _bfr(_dot(t, _bfr(w2_ref[...])) + d * b2_ref[...])
    o_ref[...] = _dot(h, _bfr(wo_ref[...])) + bo_ref[...]


_full = lambda shape: pl.BlockSpec(shape, lambda i: (0,) * len(shape))

_tc_pre = pl.pallas_call(
    _tc_pre_body,
    grid=(NP // BLK,),
    in_specs=[
        pl.BlockSpec((BLK, D), lambda i: (i, 0)),
        _full((2 * D, D)),
        _full((1, D)),
    ],
    out_specs=[pl.BlockSpec((BLK, D), lambda i: (i, 0))] * 2,
    out_shape=[jax.ShapeDtypeStruct((NP, D), jnp.float32)] * 2,
)

_tc_mid = pl.pallas_call(
    _tc_mid_body,
    grid=(NP // BLK,),
    in_specs=[
        pl.BlockSpec((2, BLK, D), lambda i: (0, i, 0)),
        pl.BlockSpec((2, BLK, DEGW), lambda i: (0, i, 0)),
        _full((D, D)),
        _full((1, D)),
        _full((2 * D, D)),
        _full((1, D)),
    ],
    out_specs=[pl.BlockSpec((BLK, D), lambda i: (i, 0))] * 2,
    out_shape=[jax.ShapeDtypeStruct((NP, D), jnp.float32)] * 2,
)

_tc_fin = pl.pallas_call(
    _tc_fin_body,
    grid=(NP // BLK,),
    in_specs=[
        pl.BlockSpec((2, BLK, D), lambda i: (0, i, 0)),
        pl.BlockSpec((2, BLK, DEGW), lambda i: (0, i, 0)),
        _full((D, D)),
        _full((1, D)),
        _full((D, D)),
        _full((1, D)),
    ],
    out_specs=pl.BlockSpec((BLK, D), lambda i: (i, 0)),
    out_shape=jax.ShapeDtypeStruct((NP, D), jnp.float32),
)


def kernel(x, edge_index, W1_0, b1_0, W2_0, b2_0, W1_1, b1_1, W2_1, b2_1,
           W1_2, b1_2, W2_2, b2_2, W1_3, b1_3, W2_3, b2_3, Wout, bout):
    sc_edge_layer, sc_degree = _sc_kernels()

    pad_e = EP - NEDGES
    src = jnp.concatenate(
        [edge_index[0], jnp.full((pad_e,), NNODES, jnp.int32)])
    dst = jnp.concatenate(
        [edge_index[1], jnp.full((pad_e,), NNODES, jnp.int32)])
    x_pad = jnp.pad(x, ((0, NP - NNODES), (0, 0)))
    wo_pad = jnp.pad(Wout, ((0, 0), (0, D - Wout.shape[1])))
    bo_pad = jnp.pad(bout, (0, D - bout.shape[0])).reshape(1, D)

    degp = sc_degree(dst)
    a, b = _tc_pre(x_pad, W1_0, b1_0.reshape(1, D))
    layers = [(W2_0, b2_0, W1_1, b1_1), (W2_1, b2_1, W1_2, b1_2),
              (W2_2, b2_2, W1_3, b1_3)]
    for (W2, b2, W1n, b1n) in layers:
        tp = sc_edge_layer(src, dst, a, b)
        a, b = _tc_mid(tp, degp, W2, b2.reshape(1, D), W1n, b1n.reshape(1, D))
    tp = sc_edge_layer(src, dst, a, b)
    out = _tc_fin(tp, degp, W2_3, b2_3.reshape(1, D), wo_pad, bo_pad)
    return out[:NNODES, :Wout.shape[1]]


# speedup vs baseline: 4.1876x; 1.2539x over previous
import functools

import jax
import jax.numpy as jnp
from jax import lax
from jax.experimental import pallas as pl
from jax.experimental.pallas import tpu as pltpu
from jax.experimental.pallas import tpu_sc as plsc

D = 128
NNODES = 10000
NP = 10240
NEDGES = 320000
C = 40
NW = 32
CPW = (NEDGES + NW * C - 1) // (NW * C)
EP = NW * C * CPW
RPT = NP // 16
BLK = 1024
DEGW = 16


@functools.cache
def _sc_kernels():
    mesh = plsc.VectorSubcoreMesh(core_axis_name="c", subcore_axis_name="s")

    @functools.partial(
        pl.kernel,
        mesh=mesh,
        out_type=jax.ShapeDtypeStruct((2, NP, D), jnp.float32),
        scratch_types=[
            pltpu.VMEM((C,), jnp.int32),
            pltpu.VMEM((C,), jnp.int32),
            pltpu.VMEM((C,), jnp.int32),
            pltpu.VMEM((C,), jnp.int32),
            pltpu.VMEM((C, D), jnp.float32),
            pltpu.VMEM((C, D), jnp.float32),
            pltpu.VMEM((C, D), jnp.float32),
            pltpu.VMEM((C, D), jnp.float32),
            pltpu.VMEM_SHARED((NP, D), jnp.float32),
            pltpu.SemaphoreType.DMA,
            pltpu.SemaphoreType.DMA,
            pltpu.SemaphoreType.DMA,
            pltpu.SemaphoreType.DMA,
        ],
    )
    def sc_edge_layer(src_hbm, dst_hbm, a_hbm, b_hbm, out_hbm,
                      dsti0, dsti1, srci0, srci1, ga0, ga1, gb0, gb1,
                      acc, s_a0, s_b0, s_a1, s_b1):
        cid = lax.axis_index("c")
        sid = lax.axis_index("s")
        wid = cid * 16 + sid
        sems = ((s_a0, s_b0), (s_a1, s_b1))
        idxs = ((dsti0, srci0), (dsti1, srci1))
        gas = (ga0, ga1)
        gbs = (gb0, gb1)

        def zrow(i, _):
            for j in range(D // 16):
                ga0[i, pl.ds(j * 16, 16)] = jnp.zeros((16,), jnp.float32)
            return 0
        lax.fori_loop(0, C, zrow, 0)
        for k in range(RPT // C):
            pltpu.sync_copy(ga0, acc.at[pl.ds(sid * RPT + k * C, C)])
        plsc.subcore_barrier()

        def issue(j, buf):
            dsti, srci = idxs[buf]
            sa, sb = sems[buf]
            base = pl.multiple_of((wid * CPW + j) * C, 8)
            pltpu.sync_copy(dst_hbm.at[pl.ds(base, C)], dsti)
            pltpu.sync_copy(src_hbm.at[pl.ds(base, C)], srci)
            pltpu.async_copy(a_hbm.at[dsti], gas[buf], sa)
            pltpu.async_copy(b_hbm.at[srci], gbs[buf], sb)

        def wait(buf):
            dsti, srci = idxs[buf]
            sa, sb = sems[buf]
            pltpu.make_async_copy(a_hbm.at[dsti], gas[buf], sa).wait()
            pltpu.make_async_copy(b_hbm.at[srci], gbs[buf], sb).wait()

        def compute_scatter(buf):
            ga = gas[buf]
            gb = gbs[buf]
            def row(e, _):
                for jj in range(D // 16):
                    xv = (ga[e, pl.ds(jj * 16, 16)]
                          + gb[e, pl.ds(jj * 16, 16)])
                    tv = 1.0 - 2.0 / (1.0 + jnp.exp(xv + xv))
                    ti = lax.bitcast_convert_type(tv, jnp.int32)
                    ti = (ti + 32767 + ((ti >> 16) & 1)) & jnp.int32(-65536)
                    ga[e, pl.ds(jj * 16, 16)] = lax.bitcast_convert_type(
                        ti, jnp.float32)
                return 0
            lax.fori_loop(0, C, row, 0)
            pltpu.sync_copy(ga, acc.at[idxs[buf][0]], add=True)

        issue(0, 0)
        issue(1, 1)

        def pair(p, _):
            wait(0)
            compute_scatter(0)
            issue(2 * p + 2, 0)
            wait(1)
            compute_scatter(1)
            issue(2 * p + 3, 1)
            return 0
        lax.fori_loop(0, CPW // 2 - 1, pair, 0)
        wait(0)
        compute_scatter(0)
        wait(1)
        compute_scatter(1)
        plsc.subcore_barrier()

        for k in range(RPT // C):
            r0 = sid * RPT + k * C
            pltpu.sync_copy(acc.at[pl.ds(r0, C)], ga0)
            pltpu.sync_copy(ga0, out_hbm.at[cid, pl.ds(r0, C)])

    @functools.partial(
        pl.kernel,
        mesh=mesh,
        out_type=jax.ShapeDtypeStruct((2, NP, DEGW), jnp.float32),
        scratch_types=[
            pltpu.VMEM((C,), jnp.int32),
            pltpu.VMEM((C, DEGW), jnp.float32),
            pltpu.VMEM((C, DEGW), jnp.float32),
            pltpu.VMEM_SHARED((NP, DEGW), jnp.float32),
        ],
    )
    def sc_degree(dst_hbm, out_hbm, dsti, ones, zbuf, acc):
        cid = lax.axis_index("c")
        sid = lax.axis_index("s")
        wid = cid * 16 + sid

        def fill(i, _):
            ones[i, pl.ds(0, 16)] = jnp.ones((16,), jnp.float32)
            zbuf[i, pl.ds(0, 16)] = jnp.zeros((16,), jnp.float32)
            return 0
        lax.fori_loop(0, C, fill, 0)
        for k in range(RPT // C):
            pltpu.sync_copy(zbuf, acc.at[pl.ds(sid * RPT + k * C, C)])
        plsc.subcore_barrier()

        def chunk(k, _):
            base = pl.multiple_of((wid * CPW + k) * C, 8)
            pltpu.sync_copy(dst_hbm.at[pl.ds(base, C)], dsti)
            pltpu.sync_copy(ones, acc.at[dsti], add=True)
            return 0
        lax.fori_loop(0, CPW, chunk, 0)
        plsc.subcore_barrier()

        for k in range(RPT // C):
            r0 = sid * RPT + k * C
            pltpu.sync_copy(acc.at[pl.ds(r0, C)], zbuf)
            pltpu.sync_copy(zbuf, out_hbm.at[cid, pl.ds(r0, C)])

    return sc_edge_layer, sc_degree


def _dot(a, b):
    return jnp.dot(a, b, preferred_element_type=jnp.float32,
                   precision=lax.Precision.HIGHEST)


def _bfr(v):
    return v.astype(jnp.bfloat16).astype(jnp.float32)


def _tc_pre_body(x_ref, w1_ref, b1_ref, a_ref, b_ref):
    w1 = _bfr(w1_ref[...])
    wb = w1[D:]
    wa = w1[:D] - wb
    xb = _bfr(x_ref[...])
    a_ref[...] = _dot(xb, wa) + b1_ref[...]
    b_ref[...] = _dot(xb, wb)


def _tc_mid_body(tp_ref, dp_ref, w2_ref, b2_ref, w1n_ref, b1n_ref, a_ref, b_ref):
    t = tp_ref[0] + tp_ref[1]
    d = (dp_ref[0] + dp_ref[1])[:, 0:1]
    h = _bfr(_dot(t, _bfr(w2_ref[...])) + d * b2_ref[...])
    w1n = _bfr(w1n_ref[...])
    wb = w1n[D:]
    wa = w1n[:D] - wb
    a_ref[...] = _dot(h, wa) + b1n_ref[...]
    b_ref[...] = _dot(h, wb)


def _tc_fin_body(tp_ref, dp_ref, w2_ref, b2_ref, wo_ref, bo_ref, o_ref):
    t = tp_ref[0] + tp_ref[1]
    d = (dp_ref[0] + dp_ref[1])[:, 0:1]
    h = _bfr(_dot(t, _bfr(w2_ref[...])) + d * b2_ref[...])
    o_ref[...] = _dot(h, _bfr(wo_ref[...])) + bo_ref[...]


_full = lambda shape: pl.BlockSpec(shape, lambda i: (0,) * len(shape))

_tc_pre = pl.pallas_call(
    _tc_pre_body,
    grid=(NP // BLK,),
    in_specs=[
        pl.BlockSpec((BLK, D), lambda i: (i, 0)),
        _full((2 * D, D)),
        _full((1, D)),
    ],
    out_specs=[pl.BlockSpec((BLK, D), lambda i: (i, 0))] * 2,
    out_shape=[jax.ShapeDtypeStruct((NP, D), jnp.float32)] * 2,
)

_tc_mid = pl.pallas_call(
    _tc_mid_body,
    grid=(NP // BLK,),
    in_specs=[
        pl.BlockSpec((2, BLK, D), lambda i: (0, i, 0)),
        pl.BlockSpec((2, BLK, DEGW), lambda i: (0, i, 0)),
        _full((D, D)),
        _full((1, D)),
        _full((2 * D, D)),
        _full((1, D)),
    ],
    out_specs=[pl.BlockSpec((BLK, D), lambda i: (i, 0))] * 2,
    out_shape=[jax.ShapeDtypeStruct((NP, D), jnp.float32)] * 2,
)

_tc_fin = pl.pallas_call(
    _tc_fin_body,
    grid=(NP // BLK,),
    in_specs=[
        pl.BlockSpec((2, BLK, D), lambda i: (0, i, 0)),
        pl.BlockSpec((2, BLK, DEGW), lambda i: (0, i, 0)),
        _full((D, D)),
        _full((1, D)),
        _full((D, D)),
        _full((1, D)),
    ],
    out_specs=pl.BlockSpec((BLK, D), lambda i: (i, 0)),
    out_shape=jax.ShapeDtypeStruct((NP, D), jnp.float32),
)


def kernel(x, edge_index, W1_0, b1_0, W2_0, b2_0, W1_1, b1_1, W2_1, b2_1,
           W1_2, b1_2, W2_2, b2_2, W1_3, b1_3, W2_3, b2_3, Wout, bout):
    sc_edge_layer, sc_degree = _sc_kernels()

    pad_e = EP - NEDGES
    src = jnp.concatenate(
        [edge_index[0], jnp.full((pad_e,), NNODES, jnp.int32)])
    dst = jnp.concatenate(
        [edge_index[1], jnp.full((pad_e,), NNODES, jnp.int32)])
    x_pad = jnp.pad(x, ((0, NP - NNODES), (0, 0)))
    wo_pad = jnp.pad(Wout, ((0, 0), (0, D - Wout.shape[1])))
    bo_pad = jnp.pad(bout, (0, D - bout.shape[0])).reshape(1, D)

    degp = sc_degree(dst)
    a, b = _tc_pre(x_pad, W1_0, b1_0.reshape(1, D))
    layers = [(W2_0, b2_0, W1_1, b1_1), (W2_1, b2_1, W1_2, b1_2),
              (W2_2, b2_2, W1_3, b1_3)]
    for (W2, b2, W1n, b1n) in layers:
        tp = sc_edge_layer(src, dst, a, b)
        a, b = _tc_mid(tp, degp, W2, b2.reshape(1, D), W1n, b1n.reshape(1, D))
    tp = sc_edge_layer(src, dst, a, b)
    out = _tc_fin(tp, degp, W2_3, b2_3.reshape(1, D), wo_pad, bo_pad)
    return out[:NNODES, :Wout.shape[1]]


# speedup vs baseline: 4.3807x; 1.0461x over previous
import functools

import jax
import jax.numpy as jnp
from jax import lax
from jax.experimental import pallas as pl
from jax.experimental.pallas import tpu as pltpu
from jax.experimental.pallas import tpu_sc as plsc

D = 128
NNODES = 10000
NP = 10240
NEDGES = 320000
C = 64
NW = 32
CPW = 158
EP = NW * C * CPW
RPT = NP // 16
BLK = 1024
DEGW = 16


@functools.cache
def _sc_kernels():
    mesh = plsc.VectorSubcoreMesh(core_axis_name="c", subcore_axis_name="s")

    @functools.partial(
        pl.kernel,
        mesh=mesh,
        out_type=jax.ShapeDtypeStruct((2, NP, D), jnp.float32),
        scratch_types=[
            pltpu.VMEM((C,), jnp.int32),
            pltpu.VMEM((C,), jnp.int32),
            pltpu.VMEM((C,), jnp.int32),
            pltpu.VMEM((C,), jnp.int32),
            pltpu.VMEM((C, D), jnp.float32),
            pltpu.VMEM((C, D), jnp.float32),
            pltpu.VMEM((C, D), jnp.float32),
            pltpu.VMEM((C, D), jnp.float32),
            pltpu.VMEM_SHARED((NP, D), jnp.float32),
            pltpu.SemaphoreType.DMA,
            pltpu.SemaphoreType.DMA,
            pltpu.SemaphoreType.DMA,
            pltpu.SemaphoreType.DMA,
        ],
    )
    def sc_edge_layer(src_hbm, dst_hbm, a_hbm, b_hbm, out_hbm,
                      dsti0, dsti1, srci0, srci1, ga0, ga1, gb0, gb1,
                      acc, s_a0, s_b0, s_a1, s_b1):
        cid = lax.axis_index("c")
        sid = lax.axis_index("s")
        wid = cid * 16 + sid
        sems = ((s_a0, s_b0), (s_a1, s_b1))
        idxs = ((dsti0, srci0), (dsti1, srci1))
        gas = (ga0, ga1)
        gbs = (gb0, gb1)

        def zrow(i, _):
            for j in range(D // 16):
                ga0[i, pl.ds(j * 16, 16)] = jnp.zeros((16,), jnp.float32)
            return 0
        lax.fori_loop(0, C, zrow, 0)
        for k in range(RPT // C):
            pltpu.sync_copy(ga0, acc.at[pl.ds(sid * RPT + k * C, C)])
        plsc.subcore_barrier()

        def issue(j, buf):
            dsti, srci = idxs[buf]
            sa, sb = sems[buf]
            base = pl.multiple_of((wid * CPW + j) * C, 8)
            pltpu.sync_copy(dst_hbm.at[pl.ds(base, C)], dsti)
            pltpu.sync_copy(src_hbm.at[pl.ds(base, C)], srci)
            pltpu.async_copy(a_hbm.at[dsti], gas[buf], sa)
            pltpu.async_copy(b_hbm.at[srci], gbs[buf], sb)

        def wait(buf):
            dsti, srci = idxs[buf]
            sa, sb = sems[buf]
            pltpu.make_async_copy(a_hbm.at[dsti], gas[buf], sa).wait()
            pltpu.make_async_copy(b_hbm.at[srci], gbs[buf], sb).wait()

        def compute_scatter(buf):
            ga = gas[buf]
            gb = gbs[buf]
            def row(e, _):
                for jj in range(D // 16):
                    xv = (ga[e, pl.ds(jj * 16, 16)]
                          + gb[e, pl.ds(jj * 16, 16)])
                    tv = 1.0 - 2.0 / (1.0 + jnp.exp(xv + xv))
                    ti = lax.bitcast_convert_type(tv, jnp.int32)
                    ti = (ti + 32767 + ((ti >> 16) & 1)) & jnp.int32(-65536)
                    ga[e, pl.ds(jj * 16, 16)] = lax.bitcast_convert_type(
                        ti, jnp.float32)
                return 0
            lax.fori_loop(0, C, row, 0)
            pltpu.sync_copy(ga, acc.at[idxs[buf][0]], add=True)

        issue(0, 0)
        issue(1, 1)

        def pair(p, _):
            wait(0)
            compute_scatter(0)
            issue(2 * p + 2, 0)
            wait(1)
            compute_scatter(1)
            issue(2 * p + 3, 1)
            return 0
        lax.fori_loop(0, CPW // 2 - 1, pair, 0)
        wait(0)
        compute_scatter(0)
        wait(1)
        compute_scatter(1)
        plsc.subcore_barrier()

        for k in range(RPT // C):
            r0 = sid * RPT + k * C
            pltpu.sync_copy(acc.at[pl.ds(r0, C)], ga0)
            pltpu.sync_copy(ga0, out_hbm.at[cid, pl.ds(r0, C)])

    @functools.partial(
        pl.kernel,
        mesh=mesh,
        out_type=jax.ShapeDtypeStruct((2, NP, DEGW), jnp.float32),
        scratch_types=[
            pltpu.VMEM((C,), jnp.int32),
            pltpu.VMEM((C, DEGW), jnp.float32),
            pltpu.VMEM((C, DEGW), jnp.float32),
            pltpu.VMEM_SHARED((NP, DEGW), jnp.float32),
        ],
    )
    def sc_degree(dst_hbm, out_hbm, dsti, ones, zbuf, acc):
        cid = lax.axis_index("c")
        sid = lax.axis_index("s")
        wid = cid * 16 + sid

        def fill(i, _):
            ones[i, pl.ds(0, 16)] = jnp.ones((16,), jnp.float32)
            zbuf[i, pl.ds(0, 16)] = jnp.zeros((16,), jnp.float32)
            return 0
        lax.fori_loop(0, C, fill, 0)
        for k in range(RPT // C):
            pltpu.sync_copy(zbuf, acc.at[pl.ds(sid * RPT + k * C, C)])
        plsc.subcore_barrier()

        def chunk(k, _):
            base = pl.multiple_of((wid * CPW + k) * C, 8)
            pltpu.sync_copy(dst_hbm.at[pl.ds(base, C)], dsti)
            pltpu.sync_copy(ones, acc.at[dsti], add=True)
            return 0
        lax.fori_loop(0, CPW, chunk, 0)
        plsc.subcore_barrier()

        for k in range(RPT // C):
            r0 = sid * RPT + k * C
            pltpu.sync_copy(acc.at[pl.ds(r0, C)], zbuf)
            pltpu.sync_copy(zbuf, out_hbm.at[cid, pl.ds(r0, C)])

    return sc_edge_layer, sc_degree


def _dot(a, b):
    return jnp.dot(a, b, preferred_element_type=jnp.float32,
                   precision=lax.Precision.HIGHEST)


def _bfr(v):
    return v.astype(jnp.bfloat16).astype(jnp.float32)


def _tc_pre_body(x_ref, w1_ref, b1_ref, a_ref, b_ref):
    w1 = _bfr(w1_ref[...])
    wb = w1[D:]
    wa = w1[:D] - wb
    xb = _bfr(x_ref[...])
    a_ref[...] = _dot(xb, wa) + b1_ref[...]
    b_ref[...] = _dot(xb, wb)


def _tc_mid_body(tp_ref, dp_ref, w2_ref, b2_ref, w1n_ref, b1n_ref, a_ref, b_ref):
    t = tp_ref[0] + tp_ref[1]
    d = (dp_ref[0] + dp_ref[1])[:, 0:1]
    h = _bfr(_dot(t, _bfr(w2_ref[...])) + d * b2_ref[...])
    w1n = _bfr(w1n_ref[...])
    wb = w1n[D:]
    wa = w1n[:D] - wb
    a_ref[...] = _dot(h, wa) + b1n_ref[...]
    b_ref[...] = _dot(h, wb)


def _tc_fin_body(tp_ref, dp_ref, w2_ref, b2_ref, wo_ref, bo_ref, o_ref):
    t = tp_ref[0] + tp_ref[1]
    d = (dp_ref[0] + dp_ref[1])[:, 0:1]
    h = _bfr(_dot(t, _bfr(w2_ref[...])) + d * b2_ref[...])
    o_ref[...] = _dot(h, _bfr(wo_ref[...])) + bo_ref[...]


_full = lambda shape: pl.BlockSpec(shape, lambda i: (0,) * len(shape))

_tc_pre = pl.pallas_call(
    _tc_pre_body,
    grid=(NP // BLK,),
    in_specs=[
        pl.BlockSpec((BLK, D), lambda i: (i, 0)),
        _full((2 * D, D)),
        _full((1, D)),
    ],
    out_specs=[pl.BlockSpec((BLK, D), lambda i: (i, 0))] * 2,
    out_shape=[jax.ShapeDtypeStruct((NP, D), jnp.float32)] * 2,
)

_tc_mid = pl.pallas_call(
    _tc_mid_body,
    grid=(NP // BLK,),
    in_specs=[
        pl.BlockSpec((2, BLK, D), lambda i: (0, i, 0)),
        pl.BlockSpec((2, BLK, DEGW), lambda i: (0, i, 0)),
        _full((D, D)),
        _full((1, D)),
        _full((2 * D, D)),
        _full((1, D)),
    ],
    out_specs=[pl.BlockSpec((BLK, D), lambda i: (i, 0))] * 2,
    out_shape=[jax.ShapeDtypeStruct((NP, D), jnp.float32)] * 2,
)

_tc_fin = pl.pallas_call(
    _tc_fin_body,
    grid=(NP // BLK,),
    in_specs=[
        pl.BlockSpec((2, BLK, D), lambda i: (0, i, 0)),
        pl.BlockSpec((2, BLK, DEGW), lambda i: (0, i, 0)),
        _full((D, D)),
        _full((1, D)),
        _full((D, D)),
        _full((1, D)),
    ],
    out_specs=pl.BlockSpec((BLK, D), lambda i: (i, 0)),
    out_shape=jax.ShapeDtypeStruct((NP, D), jnp.float32),
)


def kernel(x, edge_index, W1_0, b1_0, W2_0, b2_0, W1_1, b1_1, W2_1, b2_1,
           W1_2, b1_2, W2_2, b2_2, W1_3, b1_3, W2_3, b2_3, Wout, bout):
    sc_edge_layer, sc_degree = _sc_kernels()

    pad_e = EP - NEDGES
    src = jnp.concatenate(
        [edge_index[0], jnp.full((pad_e,), NNODES, jnp.int32)])
    dst = jnp.concatenate(
        [edge_index[1], jnp.full((pad_e,), NNODES, jnp.int32)])
    x_pad = jnp.pad(x, ((0, NP - NNODES), (0, 0)))
    wo_pad = jnp.pad(Wout, ((0, 0), (0, D - Wout.shape[1])))
    bo_pad = jnp.pad(bout, (0, D - bout.shape[0])).reshape(1, D)

    degp = sc_degree(dst)
    a, b = _tc_pre(x_pad, W1_0, b1_0.reshape(1, D))
    layers = [(W2_0, b2_0, W1_1, b1_1), (W2_1, b2_1, W1_2, b1_2),
              (W2_2, b2_2, W1_3, b1_3)]
    for (W2, b2, W1n, b1n) in layers:
        tp = sc_edge_layer(src, dst, a, b)
        a, b = _tc_mid(tp, degp, W2, b2.reshape(1, D), W1n, b1n.reshape(1, D))
    tp = sc_edge_layer(src, dst, a, b)
    out = _tc_fin(tp, degp, W2_3, b2_3.reshape(1, D), wo_pad, bo_pad)
    return out[:NNODES, :Wout.shape[1]]
